# Initial kernel scaffold; baseline (speedup 1.0000x reference)
import jax
import jax.numpy as jnp
from jax import lax
from jax.experimental import pallas as pl
from jax.experimental.pallas import tpu as pltpu
from jax.experimental.pallas import tpu_sc as plsc

_INFO = plsc.get_sparse_core_info()
_NC = _INFO.num_cores
_NS = _INFO.num_subcores
_NW = _NC * _NS
_L = _INFO.num_lanes

_N = 4 * 160 * 160 * 160
_PER_W = _N // _NW
_CHUNK = 25_600
_STEPS = _PER_W // _CHUNK
_NBUF = 2
_LUT_SIZE = 61


def _body(labels_hbm, lut_hbm, out_hbm, lut_v, in_bufs, out_bufs,
          in_sems, out_sems):
    wid = lax.axis_index("s") * _NC + lax.axis_index("c")
    base = wid * _PER_W

    pltpu.sync_copy(lut_hbm, lut_v)

    def in_copy(g, b):
        return pltpu.make_async_copy(
            labels_hbm.at[pl.ds(base + g * _CHUNK, _CHUNK)], in_bufs[b],
            in_sems[b])

    def out_copy(g, b):
        return pltpu.make_async_copy(
            out_bufs[b], out_hbm.at[pl.ds(base + g * _CHUNK, _CHUNK)],
            out_sems[b])

    for b in range(_NBUF):
        in_copy(b, b).start()

    for g in range(_STEPS):
        b = g % _NBUF
        in_copy(g, b).wait()

        def step(i, _, b=b):
            idx = in_bufs[b][pl.ds(i * _L, _L)]
            out_bufs[b][pl.ds(i * _L, _L)] = plsc.load_gather(lut_v, [idx])
            return 0

        lax.fori_loop(0, _CHUNK // _L, step, 0, unroll=8)

        if g >= _NBUF:
            out_copy(g - _NBUF, b).wait()
        out_copy(g, b).start()
        if g + _NBUF < _STEPS:
            in_copy(g + _NBUF, b).start()

    for g in range(_STEPS - _NBUF, _STEPS):
        out_copy(g, g % _NBUF).wait()


def kernel(labels, lut):
    flat = labels.reshape(_N).astype(jnp.int32)
    run = pl.kernel(
        _body,
        out_type=jax.ShapeDtypeStruct((_N,), jnp.int32),
        mesh=plsc.VectorSubcoreMesh(core_axis_name="c", subcore_axis_name="s"),
        scratch_types=[
            pltpu.VMEM((_LUT_SIZE,), jnp.int32),
            [pltpu.VMEM((_CHUNK,), jnp.int32) for _ in range(_NBUF)],
            [pltpu.VMEM((_CHUNK,), jnp.int32) for _ in range(_NBUF)],
            [pltpu.SemaphoreType.DMA for _ in range(_NBUF)],
            [pltpu.SemaphoreType.DMA for _ in range(_NBUF)],
        ],
    )
    return run(flat, lut.astype(jnp.int32)).reshape(labels.shape)


# speedup vs baseline: 327.6728x; 327.6728x over previous
import jax
import jax.numpy as jnp
from jax import lax
from jax.experimental import pallas as pl
from jax.experimental.pallas import tpu as pltpu
from jax.experimental.pallas import tpu_sc as plsc

_INFO = plsc.get_sparse_core_info()
_NC = _INFO.num_cores
_NS = _INFO.num_subcores
_NW = _NC * _NS
_L = _INFO.num_lanes

_N = 4 * 160 * 160 * 160
_PER_W = _N // _NW
_CHUNK = 25_600
_STEPS = _PER_W // _CHUNK
_NBUF = 2
_LUT_SIZE = 61


def _body(labels_hbm, lut_hbm, out_hbm, lut_v, in_bufs, out_bufs,
          in_sems, out_sems):
    wid = lax.axis_index("s") * _NC + lax.axis_index("c")
    base = wid * _PER_W

    pltpu.sync_copy(lut_hbm, lut_v)

    def in_copy(g, b):
        return pltpu.make_async_copy(
            labels_hbm.at[pl.ds(base + g * _CHUNK, _CHUNK)], in_bufs[b],
            in_sems[b])

    def out_copy(g, b):
        return pltpu.make_async_copy(
            out_bufs[b], out_hbm.at[pl.ds(base + g * _CHUNK, _CHUNK)],
            out_sems[b])

    for b in range(_NBUF):
        in_copy(b, b).start()

    for g in range(_STEPS):
        b = g % _NBUF
        in_copy(g, b).wait()

        def step(i, _, b=b):
            idx = in_bufs[b][pl.ds(i * _L, _L)]
            out_bufs[b][pl.ds(i * _L, _L)] = plsc.load_gather(lut_v, [idx])
            return 0

        lax.fori_loop(0, _CHUNK // _L, step, 0, unroll=8)

        if g >= _NBUF:
            out_copy(g - _NBUF, b).wait()
        out_copy(g, b).start()
        if g + _NBUF < _STEPS:
            in_copy(g + _NBUF, b).start()

    for g in range(_STEPS - _NBUF, _STEPS):
        out_copy(g, g % _NBUF).wait()


def kernel(labels, lut):
    flat = labels.reshape(_N).astype(jnp.int32)
    run = pl.kernel(
        _body,
        out_type=jax.ShapeDtypeStruct((_N,), jnp.int32),
        mesh=plsc.VectorSubcoreMesh(core_axis_name="c", subcore_axis_name="s"),
        scratch_types=[
            pltpu.VMEM((_LUT_SIZE,), jnp.int32),
            [pltpu.VMEM((_CHUNK,), jnp.int32) for _ in range(_NBUF)],
            [pltpu.VMEM((_CHUNK,), jnp.int32) for _ in range(_NBUF)],
            [pltpu.SemaphoreType.DMA for _ in range(_NBUF)],
            [pltpu.SemaphoreType.DMA for _ in range(_NBUF)],
        ],
        compiler_params=pltpu.CompilerParams(needs_layout_passes=False),
    )
    return run(flat, lut.astype(jnp.int32)).reshape(labels.shape)


# speedup vs baseline: 608.5919x; 1.8573x over previous
import jax
import jax.numpy as jnp
from jax import lax
from jax.experimental import pallas as pl
from jax.experimental.pallas import tpu as pltpu
from jax.experimental.pallas import tpu_sc as plsc

_INFO = plsc.get_sparse_core_info()
_NC = _INFO.num_cores
_NS = _INFO.num_subcores
_NW = _NC * _NS
_L = _INFO.num_lanes

_N = 4 * 160 * 160 * 160
_PER_W = _N // _NW
_CHUNK = 25_600
_STEPS = _PER_W // _CHUNK
_NBUF = 2
_K = 16
_LUT_SIZE = 61


def _body(labels_hbm, lut_hbm, out_hbm, lut_v, in_bufs, out_bufs,
          in_sems, out_sems):
    wid = lax.axis_index("s") * _NC + lax.axis_index("c")
    base = wid * _PER_W

    pltpu.sync_copy(lut_hbm, lut_v)

    def in_copy(g, b):
        return pltpu.make_async_copy(
            labels_hbm.at[pl.ds(base + g * _CHUNK, _CHUNK)], in_bufs[b],
            in_sems[b])

    def out_copy(g, b):
        return pltpu.make_async_copy(
            out_bufs[b], out_hbm.at[pl.ds(base + g * _CHUNK, _CHUNK)],
            out_sems[b])

    for b in range(_NBUF):
        in_copy(b, b).start()

    for g in range(_STEPS):
        b = g % _NBUF
        in_copy(g, b).wait()

        def step(i, _, b=b):
            base_i = i * (_K * _L)
            idxs = [in_bufs[b][pl.ds(base_i + k * _L, _L)] for k in range(_K)]
            vals = [plsc.load_gather(lut_v, [ix]) for ix in idxs]
            for k in range(_K):
                out_bufs[b][pl.ds(base_i + k * _L, _L)] = vals[k]
            return 0

        lax.fori_loop(0, _CHUNK // (_K * _L), step, 0, unroll=1)

        if g >= _NBUF:
            out_copy(g - _NBUF, b).wait()
        out_copy(g, b).start()
        if g + _NBUF < _STEPS:
            in_copy(g + _NBUF, b).start()

    for g in range(_STEPS - _NBUF, _STEPS):
        out_copy(g, g % _NBUF).wait()


def kernel(labels, lut):
    flat = labels.reshape(_N).astype(jnp.int32)
    run = pl.kernel(
        _body,
        out_type=jax.ShapeDtypeStruct((_N,), jnp.int32),
        mesh=plsc.VectorSubcoreMesh(core_axis_name="c", subcore_axis_name="s"),
        scratch_types=[
            pltpu.VMEM((_LUT_SIZE,), jnp.int32),
            [pltpu.VMEM((_CHUNK,), jnp.int32) for _ in range(_NBUF)],
            [pltpu.VMEM((_CHUNK,), jnp.int32) for _ in range(_NBUF)],
            [pltpu.SemaphoreType.DMA for _ in range(_NBUF)],
            [pltpu.SemaphoreType.DMA for _ in range(_NBUF)],
        ],
        compiler_params=pltpu.CompilerParams(needs_layout_passes=False),
    )
    return run(flat, lut.astype(jnp.int32)).reshape(labels.shape)


# speedup vs baseline: 1590.2974x; 2.6131x over previous
import jax
import jax.numpy as jnp
from jax import lax
from jax.experimental import pallas as pl
from jax.experimental.pallas import tpu as pltpu
from jax.experimental.pallas import tpu_sc as plsc

_INFO = plsc.get_sparse_core_info()
_NC = _INFO.num_cores
_NS = _INFO.num_subcores
_NW = _NC * _NS
_L = _INFO.num_lanes

_B, _D1, _D2, _D3 = 4, 160, 160, 160
_PLANES_PER_W = (_B * _D1) // _NW
_ROWS = 40
_CPP = _D2 // _ROWS
_NCHUNK = _PLANES_PER_W * _CPP
_NBUF = 2
_LUT_SIZE = 61


def _body(labels_hbm, lut_hbm, out_hbm, lut_v, in_bufs, out_bufs,
          in_sems, out_sems):
    wid = lax.axis_index("s") * _NC + lax.axis_index("c")
    d0 = wid // (_NW // _B)
    d1_base = (wid % (_NW // _B)) * _PLANES_PER_W

    pltpu.sync_copy(lut_hbm, lut_v)

    def in_copy(g, b):
        d1 = d1_base + (g >> 2)
        r0 = (g & (_CPP - 1)) * _ROWS
        return pltpu.make_async_copy(
            labels_hbm.at[d0, d1, pl.ds(r0, _ROWS), :], in_bufs[b],
            in_sems[b])

    def out_copy(g, b):
        d1 = d1_base + (g >> 2)
        r0 = (g & (_CPP - 1)) * _ROWS
        return pltpu.make_async_copy(
            out_bufs[b], out_hbm.at[d0, d1, pl.ds(r0, _ROWS), :],
            out_sems[b])

    for b in range(_NBUF):
        in_copy(b, b).start()

    def chunk_pair(gg, _):
        for b in range(_NBUF):
            g = gg * _NBUF + b
            in_copy(g, b).wait()

            @pl.when(g >= _NBUF)
            def _(b=b, g=g):
                out_copy(g - _NBUF, b).wait()

            def step(r, _, b=b):
                idxs = [in_bufs[b][r, pl.ds(c * _L, _L)]
                        for c in range(_D3 // _L)]
                vals = [plsc.load_gather(lut_v, [ix]) for ix in idxs]
                for c in range(_D3 // _L):
                    out_bufs[b][r, pl.ds(c * _L, _L)] = vals[c]
                return 0

            lax.fori_loop(0, _ROWS, step, 0, unroll=2)

            out_copy(g, b).start()

            @pl.when(g + _NBUF < _NCHUNK)
            def _(b=b, g=g):
                in_copy(g + _NBUF, b).start()

        return 0

    lax.fori_loop(0, _NCHUNK // _NBUF, chunk_pair, 0)

    for g in range(_NCHUNK - _NBUF, _NCHUNK):
        out_copy(g, g % _NBUF).wait()


def kernel(labels, lut):
    run = pl.kernel(
        _body,
        out_type=jax.ShapeDtypeStruct((_B, _D1, _D2, _D3), jnp.int32),
        mesh=plsc.VectorSubcoreMesh(core_axis_name="c", subcore_axis_name="s"),
        scratch_types=[
            pltpu.VMEM((_LUT_SIZE,), jnp.int32),
            [pltpu.VMEM((_ROWS, _D3), jnp.int32) for _ in range(_NBUF)],
            [pltpu.VMEM((_ROWS, _D3), jnp.int32) for _ in range(_NBUF)],
            [pltpu.SemaphoreType.DMA for _ in range(_NBUF)],
            [pltpu.SemaphoreType.DMA for _ in range(_NBUF)],
        ],
        compiler_params=pltpu.CompilerParams(needs_layout_passes=False),
    )
    return run(labels.astype(jnp.int32), lut.astype(jnp.int32))


# speedup vs baseline: 1748.5668x; 1.0995x over previous
import jax
import jax.numpy as jnp
from jax import lax
from jax.experimental import pallas as pl
from jax.experimental.pallas import tpu as pltpu
from jax.experimental.pallas import tpu_sc as plsc

_INFO = plsc.get_sparse_core_info()
_NC = _INFO.num_cores
_NS = _INFO.num_subcores
_NW = _NC * _NS
_L = _INFO.num_lanes

_B, _D1, _D2, _D3 = 4, 160, 160, 160
_PLANES_PER_W = (_B * _D1) // _NW
_ROWS = 40
_CPP = _D2 // _ROWS
_NCHUNK = _PLANES_PER_W * _CPP
_NBUF = 4
_LUT_SIZE = 61


def _body(labels_hbm, lut_hbm, out_hbm, lut_v, in_bufs, out_bufs,
          in_sems, out_sems):
    wid = lax.axis_index("s") * _NC + lax.axis_index("c")
    d0 = wid // (_NW // _B)
    d1_base = (wid % (_NW // _B)) * _PLANES_PER_W

    pltpu.sync_copy(lut_hbm, lut_v)

    def in_copy(g, b):
        d1 = d1_base + (g >> 2)
        r0 = (g & (_CPP - 1)) * _ROWS
        return pltpu.make_async_copy(
            labels_hbm.at[d0, d1, pl.ds(r0, _ROWS), :], in_bufs[b],
            in_sems[b])

    def out_copy(g, b):
        d1 = d1_base + (g >> 2)
        r0 = (g & (_CPP - 1)) * _ROWS
        return pltpu.make_async_copy(
            out_bufs[b], out_hbm.at[d0, d1, pl.ds(r0, _ROWS), :],
            out_sems[b])

    for b in range(_NBUF):
        in_copy(b, b).start()

    def chunk_pair(gg, _):
        for b in range(_NBUF):
            g = gg * _NBUF + b
            in_copy(g, b).wait()

            @pl.when(g >= _NBUF)
            def _(b=b, g=g):
                out_copy(g - _NBUF, b).wait()

            def step(r, _, b=b):
                idxs = [in_bufs[b][r, pl.ds(c * _L, _L)]
                        for c in range(_D3 // _L)]
                vals = [plsc.load_gather(lut_v, [ix]) for ix in idxs]
                for c in range(_D3 // _L):
                    out_bufs[b][r, pl.ds(c * _L, _L)] = vals[c]
                return 0

            lax.fori_loop(0, _ROWS, step, 0, unroll=2)

            out_copy(g, b).start()

            @pl.when(g + _NBUF < _NCHUNK)
            def _(b=b, g=g):
                in_copy(g + _NBUF, b).start()

        return 0

    lax.fori_loop(0, _NCHUNK // _NBUF, chunk_pair, 0)

    for g in range(_NCHUNK - _NBUF, _NCHUNK):
        out_copy(g, g % _NBUF).wait()


def kernel(labels, lut):
    run = pl.kernel(
        _body,
        out_type=jax.ShapeDtypeStruct((_B, _D1, _D2, _D3), jnp.int32),
        mesh=plsc.VectorSubcoreMesh(core_axis_name="c", subcore_axis_name="s"),
        scratch_types=[
            pltpu.VMEM((_LUT_SIZE,), jnp.int32),
            [pltpu.VMEM((_ROWS, _D3), jnp.int32) for _ in range(_NBUF)],
            [pltpu.VMEM((_ROWS, _D3), jnp.int32) for _ in range(_NBUF)],
            [pltpu.SemaphoreType.DMA for _ in range(_NBUF)],
            [pltpu.SemaphoreType.DMA for _ in range(_NBUF)],
        ],
        compiler_params=pltpu.CompilerParams(needs_layout_passes=False),
    )
    return run(labels.astype(jnp.int32), lut.astype(jnp.int32))


# speedup vs baseline: 1751.2716x; 1.0015x over previous
import jax
import jax.numpy as jnp
from jax import lax
from jax.experimental import pallas as pl
from jax.experimental.pallas import tpu as pltpu
from jax.experimental.pallas import tpu_sc as plsc

_INFO = plsc.get_sparse_core_info()
_NC = _INFO.num_cores
_NS = _INFO.num_subcores
_NW = _NC * _NS
_L = _INFO.num_lanes

_B, _D1, _D2, _D3 = 4, 160, 160, 160
_PLANES_PER_W = (_B * _D1) // _NW
_ROWS = 40
_CPP = _D2 // _ROWS
_NCHUNK = _PLANES_PER_W * _CPP
_NBUF = 5
_LUT_SIZE = 61


def _body(labels_hbm, lut_hbm, out_hbm, lut_v, in_bufs, out_bufs,
          in_sems, out_sems):
    wid = lax.axis_index("s") * _NC + lax.axis_index("c")
    d0 = wid // (_NW // _B)
    d1_base = (wid % (_NW // _B)) * _PLANES_PER_W

    pltpu.sync_copy(lut_hbm, lut_v)

    def in_copy(g, b):
        d1 = d1_base + (g >> 2)
        r0 = (g & (_CPP - 1)) * _ROWS
        return pltpu.make_async_copy(
            labels_hbm.at[d0, d1, pl.ds(r0, _ROWS), :], in_bufs[b],
            in_sems[b])

    def out_copy(g, b):
        d1 = d1_base + (g >> 2)
        r0 = (g & (_CPP - 1)) * _ROWS
        return pltpu.make_async_copy(
            out_bufs[b], out_hbm.at[d0, d1, pl.ds(r0, _ROWS), :],
            out_sems[b])

    for b in range(_NBUF):
        in_copy(b, b).start()

    def chunk_pair(gg, _):
        for b in range(_NBUF):
            g = gg * _NBUF + b
            in_copy(g, b).wait()

            @pl.when(g >= _NBUF)
            def _(b=b, g=g):
                out_copy(g - _NBUF, b).wait()

            def step(r, _, b=b):
                idxs = [in_bufs[b][r, pl.ds(c * _L, _L)]
                        for c in range(_D3 // _L)]
                vals = [plsc.load_gather(lut_v, [ix]) for ix in idxs]
                for c in range(_D3 // _L):
                    out_bufs[b][r, pl.ds(c * _L, _L)] = vals[c]
                return 0

            lax.fori_loop(0, _ROWS, step, 0, unroll=2)

            out_copy(g, b).start()

            @pl.when(g + _NBUF < _NCHUNK)
            def _(b=b, g=g):
                in_copy(g + _NBUF, b).start()

        return 0

    lax.fori_loop(0, _NCHUNK // _NBUF, chunk_pair, 0)

    for g in range(_NCHUNK - _NBUF, _NCHUNK):
        out_copy(g, g % _NBUF).wait()


def kernel(labels, lut):
    run = pl.kernel(
        _body,
        out_type=jax.ShapeDtypeStruct((_B, _D1, _D2, _D3), jnp.int32),
        mesh=plsc.VectorSubcoreMesh(core_axis_name="c", subcore_axis_name="s"),
        scratch_types=[
            pltpu.VMEM((_LUT_SIZE,), jnp.int32),
            [pltpu.VMEM((_ROWS, _D3), jnp.int32) for _ in range(_NBUF)],
            [pltpu.VMEM((_ROWS, _D3), jnp.int32) for _ in range(_NBUF)],
            [pltpu.SemaphoreType.DMA for _ in range(_NBUF)],
            [pltpu.SemaphoreType.DMA for _ in range(_NBUF)],
        ],
        compiler_params=pltpu.CompilerParams(needs_layout_passes=False),
    )
    return run(labels.astype(jnp.int32), lut.astype(jnp.int32))
